# baseline (speedup 1.0000x reference)
import functools

import jax
import jax.numpy as jnp
from jax import lax
from jax.experimental import pallas as pl
from jax.experimental.pallas import tpu as pltpu
from jax.experimental.pallas import tpu_sc as plsc

_B, _L, _D, _CLS = 8, 2048, 64, 20
_BLK_R = 256
_NBLK = _L // _BLK_R

_NC, _NS = 2, 16
_NW = _NC * _NS
_RPW = (_B * _L) // _NW
_CHUNK = 128
_NCH = _RPW // _CHUNK


def _sc_gather(table, idx):
    mesh = plsc.VectorSubcoreMesh(core_axis_name="c", subcore_axis_name="s")

    @functools.partial(
        pl.kernel,
        mesh=mesh,
        out_type=jax.ShapeDtypeStruct((_B * _L, _D), jnp.float32),
        scratch_types=[
            pltpu.VMEM((_NCH, _CHUNK), jnp.int32),
            pltpu.VMEM((_RPW, _D), jnp.float32),
            pltpu.SemaphoreType.DMA,
        ],
        compiler_params=pltpu.CompilerParams(use_tc_tiling_on_sc=False),
    )
    def k(table_hbm, idx_hbm, out_hbm, idx_v, rows_v, sem):
        wid = lax.axis_index("s") * _NC + lax.axis_index("c")
        base = wid * _RPW
        pltpu.sync_copy(idx_hbm.at[wid], idx_v)
        copies = [
            pltpu.async_copy(
                table_hbm.at[idx_v.at[j]],
                rows_v.at[pl.ds(j * _CHUNK, _CHUNK)],
                sem,
            )
            for j in range(_NCH)
        ]
        for cp in copies:
            cp.wait()
        pltpu.sync_copy(rows_v, out_hbm.at[pl.ds(base, _RPW)])

    return k(table, idx)


_NSPLIT = 4
_WS = _L // _NSPLIT


def _tc_body(gath_ref, imask_ref, memb_ref, *rest):
    a_refs = rest[:_NSPLIT]
    (w1_ref, b1_ref, w2_ref, b2_ref, wp_ref, bp_ref, out_ref) = rest[_NSPLIT:]
    b = pl.program_id(0)
    x = gath_ref[0]
    msk = imask_ref[0, 0, :]
    sig = jax.nn.sigmoid(memb_ref[...])
    f = jnp.where(msk[:, None] == 1, sig[1:2, :], sig[0:1, :])
    s1 = jnp.dot(x * f, w1_ref[...], preferred_element_type=jnp.float32)

    h = b1_ref[...]
    for j in range(_NSPLIT):
        h = h + jnp.dot(a_refs[j][0], s1[j * _WS:(j + 1) * _WS, :],
                        preferred_element_type=jnp.float32)
    h1 = 0.5 * h * (1.0 + lax.erf(h * (2.0 ** -0.5)))

    pooled = jnp.zeros((1, _D), jnp.float32)
    for j in range(_NSPLIT):
        cj = jnp.sum(a_refs[j][0], axis=0, keepdims=True)
        pooled = pooled + jnp.dot(cj, h1[j * _WS:(j + 1) * _WS, :],
                                  preferred_element_type=jnp.float32)
    pooled = jnp.dot(pooled, w2_ref[...],
                     preferred_element_type=jnp.float32) + _L * b2_ref[...]
    logits = jnp.dot(pooled, wp_ref[...],
                     preferred_element_type=jnp.float32) + bp_ref[...]
    m = jnp.max(logits, axis=1, keepdims=True)
    lse = jnp.log(jnp.sum(jnp.exp(logits - m), axis=1, keepdims=True)) + m
    out_ref[pl.ds(b, 1), :] = logits - lse


def _tc_forward(gathered, imask3, mask_embedding, paris_mat, W1, b1, W2, b2,
                Wp, bp):
    def a_spec(j):
        return pl.BlockSpec((1, _L, _WS), lambda b, j=j: (b, 0, j))

    return pl.pallas_call(
        _tc_body,
        grid=(_B,),
        in_specs=[
            pl.BlockSpec((1, _L, _D), lambda b: (b, 0, 0)),
            pl.BlockSpec((1, 1, _L), lambda b: (b, 0, 0)),
            pl.BlockSpec((2, _D), lambda b: (0, 0)),
        ] + [a_spec(j) for j in range(_NSPLIT)] + [
            pl.BlockSpec((_D, _D), lambda b: (0, 0)),
            pl.BlockSpec((1, _D), lambda b: (0, 0)),
            pl.BlockSpec((_D, _D), lambda b: (0, 0)),
            pl.BlockSpec((1, _D), lambda b: (0, 0)),
            pl.BlockSpec((_D, _CLS), lambda b: (0, 0)),
            pl.BlockSpec((1, _CLS), lambda b: (0, 0)),
        ],
        out_specs=pl.BlockSpec((_B, _CLS), lambda b: (0, 0)),
        out_shape=jax.ShapeDtypeStruct((_B, _CLS), jnp.float32),
        compiler_params=pltpu.CompilerParams(
            dimension_semantics=("arbitrary",),
        ),
    )(gathered, imask3, mask_embedding,
      *([paris_mat] * _NSPLIT),
      W1, b1, W2, b2, Wp, bp)


def kernel(words2ids, i_mask, paris_mat, w_embedding, mask_embedding,
           W1, b1, W2, b2, Wp, bp):
    idx = words2ids.astype(jnp.int32).reshape(_NW, _NCH, _CHUNK)
    gathered = _sc_gather(w_embedding, idx).reshape(_B, _L, _D)
    imask3 = i_mask.astype(jnp.int32).reshape(_B, 1, _L)
    return _tc_forward(gathered, imask3, mask_embedding, paris_mat,
                       W1, b1.reshape(1, _D), W2, b2.reshape(1, _D),
                       Wp, bp.reshape(1, _CLS))


# speedup vs baseline: 1.0774x; 1.0774x over previous
import functools

import jax
import jax.numpy as jnp
from jax import lax
from jax.experimental import pallas as pl
from jax.experimental.pallas import tpu as pltpu
from jax.experimental.pallas import tpu_sc as plsc

_B, _L, _D, _CLS = 8, 2048, 64, 20

_NC, _NS = 2, 16
_NW = _NC * _NS
_RPW = (_B * _L) // _NW
_SEG = _NW // _B
_CHUNK = 128
_NCH = _RPW // _CHUNK


def _sc_gather(table, idx):
    mesh = plsc.VectorSubcoreMesh(core_axis_name="c", subcore_axis_name="s")

    @functools.partial(
        pl.kernel,
        mesh=mesh,
        out_type=jax.ShapeDtypeStruct((_B * _L // 2, 2 * _D), jnp.float32),
        scratch_types=[
            pltpu.VMEM((_RPW,), jnp.int32),
            pltpu.VMEM((_RPW, _D), jnp.float32),
            pltpu.SemaphoreType.DMA,
        ],
        compiler_params=pltpu.CompilerParams(use_tc_tiling_on_sc=False),
    )
    def k(table_hbm, idx_hbm, out_hbm, idx_v, rows_v, sem):
        wid = lax.axis_index("s") * _NC + lax.axis_index("c")
        b = wid // _SEG
        seg = wid % _SEG
        pltpu.sync_copy(idx_hbm.at[b, pl.ds(seg * _RPW, _RPW)], idx_v)
        copies = [
            pltpu.async_copy(
                table_hbm.at[idx_v.at[pl.ds(j * _CHUNK, _CHUNK)]],
                rows_v.at[pl.ds(j * _CHUNK, _CHUNK)],
                sem,
            )
            for j in range(_NCH)
        ]
        for cp in copies:
            cp.wait()
        row0 = b * (_L // 2) + (seg % 2) * _RPW
        col0 = (seg // 2) * _D
        pltpu.sync_copy(rows_v,
                        out_hbm.at[pl.ds(row0, _RPW), pl.ds(col0, _D)])

    return k(table, idx)


def _tc_body(gath_ref, imask_ref, memb_ref, a_ref, w1_ref, b1_ref, w2_ref,
             b2_ref, wp_ref, bp_ref, out_ref):
    b = pl.program_id(0)
    x2 = gath_ref[0]
    x = jnp.concatenate([x2[:, :_D], x2[:, _D:]], axis=0)
    msk = imask_ref[0, 0, :]
    sig = jax.nn.sigmoid(memb_ref[...])
    f = jnp.where(msk[:, None] == 1, sig[1:2, :], sig[0:1, :])
    s1 = jnp.dot(x * f, w1_ref[...], preferred_element_type=jnp.float32)

    a = a_ref[0]
    h = jnp.dot(a, s1, preferred_element_type=jnp.float32) + b1_ref[...]
    h1 = 0.5 * h * (1.0 + lax.erf(h * (2.0 ** -0.5)))
    c = jnp.sum(a, axis=0, keepdims=True)

    pooled = jnp.dot(c, h1, preferred_element_type=jnp.float32)
    pooled = jnp.dot(pooled, w2_ref[...],
                     preferred_element_type=jnp.float32) + _L * b2_ref[...]
    logits = jnp.dot(pooled, wp_ref[...],
                     preferred_element_type=jnp.float32) + bp_ref[...]
    m = jnp.max(logits, axis=1, keepdims=True)
    lse = jnp.log(jnp.sum(jnp.exp(logits - m), axis=1, keepdims=True)) + m
    out_ref[pl.ds(b, 1), :] = logits - lse


def _tc_forward(gathered2, imask3, mask_embedding, paris_mat, W1, b1, W2, b2,
                Wp, bp):
    return pl.pallas_call(
        _tc_body,
        grid=(_B,),
        in_specs=[
            pl.BlockSpec((1, _L // 2, 2 * _D), lambda b: (b, 0, 0)),
            pl.BlockSpec((1, 1, _L), lambda b: (b, 0, 0)),
            pl.BlockSpec((2, _D), lambda b: (0, 0)),
            pl.BlockSpec((1, _L, _L), lambda b: (b, 0, 0)),
            pl.BlockSpec((_D, _D), lambda b: (0, 0)),
            pl.BlockSpec((1, _D), lambda b: (0, 0)),
            pl.BlockSpec((_D, _D), lambda b: (0, 0)),
            pl.BlockSpec((1, _D), lambda b: (0, 0)),
            pl.BlockSpec((_D, _CLS), lambda b: (0, 0)),
            pl.BlockSpec((1, _CLS), lambda b: (0, 0)),
        ],
        out_specs=pl.BlockSpec((_B, _CLS), lambda b: (0, 0)),
        out_shape=jax.ShapeDtypeStruct((_B, _CLS), jnp.float32),
        compiler_params=pltpu.CompilerParams(
            dimension_semantics=("arbitrary",),
        ),
    )(gathered2, imask3, mask_embedding, paris_mat, W1, b1, W2, b2, Wp, bp)


def kernel(words2ids, i_mask, paris_mat, w_embedding, mask_embedding,
           W1, b1, W2, b2, Wp, bp):
    idx = words2ids.astype(jnp.int32)
    gathered = _sc_gather(w_embedding, idx)
    gathered2 = gathered.reshape(_B, _L // 2, 2 * _D)
    imask3 = i_mask.astype(jnp.int32).reshape(_B, 1, _L)
    return _tc_forward(gathered2, imask3, mask_embedding, paris_mat,
                       W1, b1.reshape(1, _D), W2, b2.reshape(1, _D),
                       Wp, bp.reshape(1, _CLS))
